# Initial kernel scaffold; baseline (speedup 1.0000x reference)
import jax
import jax.numpy as jnp
from jax.experimental import pallas as pl


def kernel(features, boxes, scores, image_sizes, W_fc, b_fc):
    raise NotImplementedError("write your pallas kernel here")


# speedup vs baseline: 7.9065x; 7.9065x over previous
import jax
import jax.numpy as jnp
from jax.experimental import pallas as pl

PRE_NMS = 1000
POST_NMS = 300
NMS_THRESH = 0.7
OUT_SIZE = 7
NPAD = 1024
RBLK = 16
RPAD = 320
NRB = RPAD // RBLK
NBINS = OUT_SIZE * OUT_SIZE
ROWS = RBLK * NBINS


def _nms_kernel(coords_ref, keep_ref):
    c = coords_ref[0]
    x1r = c[0:1, :]
    y1r = c[1:2, :]
    x2r = c[2:3, :]
    y2r = c[3:4, :]
    x1c = jnp.transpose(x1r)
    y1c = jnp.transpose(y1r)
    x2c = jnp.transpose(x2r)
    y2c = jnp.transpose(y2r)
    area_r = (x2r - x1r) * (y2r - y1r)
    area_c = (x2c - x1c) * (y2c - y1c)
    wx = jnp.clip(jnp.minimum(x2c, x2r) - jnp.maximum(x1c, x1r), 0.0)
    wy = jnp.clip(jnp.minimum(y2c, y2r) - jnp.maximum(y1c, y1r), 0.0)
    inter = wx * wy
    iou = inter / (area_c + area_r - inter + 1e-9)
    ii = jax.lax.broadcasted_iota(jnp.int32, (NPAD, NPAD), 0)
    jj = jax.lax.broadcasted_iota(jnp.int32, (NPAD, NPAD), 1)
    sup_mat = ((iou > NMS_THRESH) & (jj > ii)).astype(jnp.float32)

    def cond(carry):
        return carry[1]

    def body(carry):
        keep, _ = carry
        hits = jnp.dot(keep, sup_mat, preferred_element_type=jnp.float32)
        new = jnp.where(hits > 0.0, 0.0, 1.0)
        return new, jnp.any(new != keep)

    keep0 = jnp.ones((1, NPAD), jnp.float32)
    keep, _ = jax.lax.while_loop(cond, body, (keep0, jnp.array(True)))
    keep_ref[0] = keep


def _roi_kernel(rois_ref, feat_ref, out_ref):
    r = rois_ref[0, 0]
    x1 = r[:, 0:1]
    y1 = r[:, 1:2]
    x2 = r[:, 2:3]
    y2 = r[:, 3:4]
    feat = feat_ref[0]
    HW = feat.shape[0]
    H = 50
    fdim = jnp.float32(H)

    rowi = jax.lax.broadcasted_iota(jnp.int32, (ROWS, 1), 0)
    pbin = ((rowi // OUT_SIZE) % OUT_SIZE).astype(jnp.float32)
    qbin = (rowi % OUT_SIZE).astype(jnp.float32)
    lane = jax.lax.broadcasted_iota(jnp.int32, (ROWS, H), 1)

    def axis_weights(lo, hi, binidx):
        binsz = (hi - lo) / OUT_SIZE
        acc = jnp.zeros((ROWS, H), jnp.float32)
        for a in (0, 1):
            frac = binidx + (a + 0.5) / 2.0
            ss = lo + frac * binsz
            v = ((ss >= -1.0) & (ss <= fdim)).astype(jnp.float32)
            sc = jnp.clip(ss, 0.0, fdim - 1.0)
            f0 = jnp.floor(sc)
            i0 = f0.astype(jnp.int32)
            i1 = jnp.minimum(i0 + 1, H - 1)
            lw = sc - f0
            hw = 1.0 - lw
            w = hw * (i0 == lane).astype(jnp.float32) \
                + lw * (i1 == lane).astype(jnp.float32)
            acc = acc + w * v
        return acc * 0.5

    ay = axis_weights(y1, y2, pbin)
    ax = axis_weights(x1, x2, qbin)

    er = jax.lax.broadcasted_iota(jnp.int32, (H, HW), 0)
    el = jax.lax.broadcasted_iota(jnp.int32, (H, HW), 1)
    eh = ((el // H) == er).astype(jnp.float32)
    ew = ((el % H) == er).astype(jnp.float32)
    amat = jnp.dot(ay, eh, preferred_element_type=jnp.float32) \
        * jnp.dot(ax, ew, preferred_element_type=jnp.float32)
    out_ref[0, 0] = jnp.dot(amat, feat, preferred_element_type=jnp.float32)


def _fc_kernel(x_ref, w_ref, b_ref, out_ref):
    k = pl.program_id(1)

    @pl.when(k == 0)
    def _init():
        out_ref[...] = jnp.broadcast_to(b_ref[...], out_ref.shape)

    out_ref[...] += jax.lax.dot_general(
        x_ref[...], w_ref[...], (((1,), (1,)), ((), ())),
        preferred_element_type=jnp.float32)


def kernel(features, boxes, scores, image_sizes, W_fc, b_fc):
    B, C, H, W = features.shape
    img_h = image_sizes[0, 0].astype(jnp.float32)
    img_w = image_sizes[0, 1].astype(jnp.float32)
    scale = H / img_h
    neg_inf = jnp.float32(-jnp.inf)

    vals, idx = jax.lax.top_k(scores, PRE_NMS)
    cand = jnp.take_along_axis(boxes, idx[..., None], axis=1)
    cx1 = jnp.clip(cand[..., 0], 0.0, img_w)
    cy1 = jnp.clip(cand[..., 1], 0.0, img_h)
    cx2 = jnp.clip(cand[..., 2], 0.0, img_w)
    cy2 = jnp.clip(cand[..., 3], 0.0, img_h)
    cand = jnp.stack([cx1, cy1, cx2, cy2], axis=-1)
    valid = ((cx2 - cx1) > 1e-3) & ((cy2 - cy1) > 1e-3)
    vals = jnp.where(valid, vals, neg_inf)
    order = jnp.argsort(-vals, axis=-1)
    cand = jnp.take_along_axis(cand, order[..., None], axis=1)
    vals = jnp.take_along_axis(vals, order, axis=1)
    valid = jnp.take_along_axis(valid, order, axis=1)

    pad = NPAD - PRE_NMS
    candp = jnp.pad(cand, ((0, 0), (0, pad), (0, 0)))
    valsp = jnp.pad(vals, ((0, 0), (0, pad)), constant_values=-jnp.inf)
    validp = jnp.pad(valid, ((0, 0), (0, pad)))

    coords = jnp.transpose(candp, (0, 2, 1))
    keep = pl.pallas_call(
        _nms_kernel,
        grid=(B,),
        in_specs=[pl.BlockSpec((1, 4, NPAD), lambda b: (b, 0, 0))],
        out_specs=pl.BlockSpec((1, 1, NPAD), lambda b: (b, 0, 0)),
        out_shape=jax.ShapeDtypeStruct((B, 1, NPAD), jnp.float32),
    )(coords)[:, 0, :]
    keepb = (keep > 0.5) & validp

    ks = jnp.where(keepb, valsp, neg_inf)
    pv, pidx = jax.lax.top_k(ks, POST_NMS)
    mask = (pv > neg_inf).astype(jnp.float32)
    props = jnp.take_along_axis(candp, pidx[..., None], axis=1) * mask[..., None]

    rois = props * scale - 0.5
    rois = jnp.pad(rois, ((0, 0), (0, RPAD - POST_NMS), (0, 0)))
    rois = jnp.repeat(rois, NBINS, axis=1).reshape(B, NRB, ROWS, 4)
    feat2 = jnp.transpose(features, (0, 2, 3, 1)).reshape(B, H * W, C)
    pooled = pl.pallas_call(
        _roi_kernel,
        grid=(B, NRB),
        in_specs=[
            pl.BlockSpec((1, 1, ROWS, 4), lambda b, rb: (b, rb, 0, 0)),
            pl.BlockSpec((1, H * W, C), lambda b, rb: (b, 0, 0)),
        ],
        out_specs=pl.BlockSpec((1, 1, ROWS, C), lambda b, rb: (b, rb, 0, 0)),
        out_shape=jax.ShapeDtypeStruct((B, NRB, ROWS, C), jnp.float32),
    )(rois, feat2)
    flat = pooled.reshape(B, RPAD, NBINS * C)[:, :POST_NMS]
    flat = flat.reshape(B * POST_NMS, NBINS * C)

    M = B * POST_NMS
    MP = 640
    K = flat.shape[1]
    N = W_fc.shape[0]
    BN, BK = 128, 896
    W_perm = W_fc.reshape(N, C, OUT_SIZE, OUT_SIZE)
    W_perm = jnp.transpose(W_perm, (0, 2, 3, 1)).reshape(N, K)
    x = jnp.pad(flat, ((0, MP - M), (0, 0)))
    b2 = b_fc.reshape(1, N)
    out = pl.pallas_call(
        _fc_kernel,
        grid=(N // BN, K // BK),
        in_specs=[
            pl.BlockSpec((MP, BK), lambda n, k: (0, k)),
            pl.BlockSpec((BN, BK), lambda n, k: (n, k)),
            pl.BlockSpec((1, BN), lambda n, k: (0, n)),
        ],
        out_specs=pl.BlockSpec((MP, BN), lambda n, k: (0, n)),
        out_shape=jax.ShapeDtypeStruct((MP, N), jnp.float32),
    )(x, W_perm, b2)
    return out[:M]


# speedup vs baseline: 10.2020x; 1.2903x over previous
import jax
import jax.numpy as jnp
from jax.experimental import pallas as pl

PRE_NMS = 1000
POST_NMS = 300
NMS_THRESH = 0.7
OUT_SIZE = 7
NPAD = 1024
RBLK = 24
RPAD = 312
NRB = RPAD // RBLK
NBINS = OUT_SIZE * OUT_SIZE
ROWS = RBLK * NBINS


def _nms_kernel(coords_ref, keep_ref):
    c = coords_ref[0]
    x1r = c[0:1, :]
    y1r = c[1:2, :]
    x2r = c[2:3, :]
    y2r = c[3:4, :]
    x1c = jnp.transpose(x1r)
    y1c = jnp.transpose(y1r)
    x2c = jnp.transpose(x2r)
    y2c = jnp.transpose(y2r)
    area_r = (x2r - x1r) * (y2r - y1r)
    area_c = (x2c - x1c) * (y2c - y1c)
    wx = jnp.clip(jnp.minimum(x2c, x2r) - jnp.maximum(x1c, x1r), 0.0)
    wy = jnp.clip(jnp.minimum(y2c, y2r) - jnp.maximum(y1c, y1r), 0.0)
    inter = wx * wy
    iou = inter / (area_c + area_r - inter + 1e-9)
    ii = jax.lax.broadcasted_iota(jnp.int32, (NPAD, NPAD), 0)
    jj = jax.lax.broadcasted_iota(jnp.int32, (NPAD, NPAD), 1)
    sup_mat = ((iou > NMS_THRESH) & (jj > ii)).astype(jnp.float32)

    def cond(carry):
        return carry[1]

    def body(carry):
        keep, _ = carry
        hits = jnp.dot(keep, sup_mat, preferred_element_type=jnp.float32)
        new = jnp.where(hits > 0.0, 0.0, 1.0)
        return new, jnp.any(new != keep)

    keep0 = jnp.ones((1, NPAD), jnp.float32)
    keep, _ = jax.lax.while_loop(cond, body, (keep0, jnp.array(True)))
    keep_ref[0] = keep


def _roi_kernel(rois_ref, feat_ref, out_ref):
    r = rois_ref[0, 0]
    x1 = r[0:1, :]
    y1 = r[1:2, :]
    x2 = r[2:3, :]
    y2 = r[3:4, :]
    feat = feat_ref[0]
    HW = feat.shape[1]
    H = 50
    fdim = jnp.float32(H)

    coli = jax.lax.broadcasted_iota(jnp.int32, (1, ROWS), 1)
    pbin = ((coli // OUT_SIZE) % OUT_SIZE).astype(jnp.float32)
    qbin = (coli % OUT_SIZE).astype(jnp.float32)
    subl = jax.lax.broadcasted_iota(jnp.int32, (H, ROWS), 0)

    def axis_weights(lo, hi, binidx):
        binsz = (hi - lo) / OUT_SIZE
        acc = jnp.zeros((H, ROWS), jnp.float32)
        for a in (0, 1):
            frac = binidx + (a + 0.5) / 2.0
            ss = lo + frac * binsz
            v = ((ss >= -1.0) & (ss <= fdim)).astype(jnp.float32)
            sc = jnp.clip(ss, 0.0, fdim - 1.0)
            f0 = jnp.floor(sc)
            i0 = f0.astype(jnp.int32)
            i1 = jnp.minimum(i0 + 1, H - 1)
            lw = sc - f0
            hw = 1.0 - lw
            w = hw * (i0 == subl).astype(jnp.float32) \
                + lw * (i1 == subl).astype(jnp.float32)
            acc = acc + w * v
        return acc * 0.5

    ay = axis_weights(y1, y2, pbin)
    ax = axis_weights(x1, x2, qbin)

    er = jax.lax.broadcasted_iota(jnp.int32, (HW, H), 0)
    el = jax.lax.broadcasted_iota(jnp.int32, (HW, H), 1)
    eh = ((er // H) == el).astype(jnp.float32)
    ew = ((er % H) == el).astype(jnp.float32)
    amat = jnp.dot(eh, ay, preferred_element_type=jnp.float32) \
        * jnp.dot(ew, ax, preferred_element_type=jnp.float32)
    out_ref[0, 0] = jnp.dot(feat, amat, preferred_element_type=jnp.float32)


def _fc_kernel(x_ref, w_ref, b_ref, out_ref):
    k = pl.program_id(1)

    @pl.when(k == 0)
    def _init():
        out_ref[...] = jnp.broadcast_to(b_ref[...], out_ref.shape)

    out_ref[...] += jax.lax.dot_general(
        x_ref[...], w_ref[...], (((1,), (1,)), ((), ())),
        preferred_element_type=jnp.float32)


def kernel(features, boxes, scores, image_sizes, W_fc, b_fc):
    B, C, H, W = features.shape
    img_h = image_sizes[0, 0].astype(jnp.float32)
    img_w = image_sizes[0, 1].astype(jnp.float32)
    scale = H / img_h
    neg_inf = jnp.float32(-jnp.inf)

    vals, idx = jax.lax.top_k(scores, PRE_NMS)
    cand = jnp.take_along_axis(boxes, idx[..., None], axis=1)
    cx1 = jnp.clip(cand[..., 0], 0.0, img_w)
    cy1 = jnp.clip(cand[..., 1], 0.0, img_h)
    cx2 = jnp.clip(cand[..., 2], 0.0, img_w)
    cy2 = jnp.clip(cand[..., 3], 0.0, img_h)
    cand = jnp.stack([cx1, cy1, cx2, cy2], axis=-1)
    valid = ((cx2 - cx1) > 1e-3) & ((cy2 - cy1) > 1e-3)
    vals = jnp.where(valid, vals, neg_inf)
    order = jnp.argsort(-vals, axis=-1)
    cand = jnp.take_along_axis(cand, order[..., None], axis=1)
    vals = jnp.take_along_axis(vals, order, axis=1)
    valid = jnp.take_along_axis(valid, order, axis=1)

    pad = NPAD - PRE_NMS
    candp = jnp.pad(cand, ((0, 0), (0, pad), (0, 0)))
    valsp = jnp.pad(vals, ((0, 0), (0, pad)), constant_values=-jnp.inf)
    validp = jnp.pad(valid, ((0, 0), (0, pad)))

    coords = jnp.transpose(candp, (0, 2, 1))
    keep = pl.pallas_call(
        _nms_kernel,
        grid=(B,),
        in_specs=[pl.BlockSpec((1, 4, NPAD), lambda b: (b, 0, 0))],
        out_specs=pl.BlockSpec((1, 1, NPAD), lambda b: (b, 0, 0)),
        out_shape=jax.ShapeDtypeStruct((B, 1, NPAD), jnp.float32),
    )(coords)[:, 0, :]
    keepb = (keep > 0.5) & validp

    ks = jnp.where(keepb, valsp, neg_inf)
    pv, pidx = jax.lax.top_k(ks, POST_NMS)
    mask = (pv > neg_inf).astype(jnp.float32)
    props = jnp.take_along_axis(candp, pidx[..., None], axis=1) * mask[..., None]

    rois = props * scale - 0.5
    rois = jnp.pad(rois, ((0, 0), (0, RPAD - POST_NMS), (0, 0)))
    rois = jnp.repeat(rois, NBINS, axis=1).reshape(B, NRB, ROWS, 4)
    rois = jnp.transpose(rois, (0, 1, 3, 2))
    feat2 = features.reshape(B, C, H * W)
    pooled = pl.pallas_call(
        _roi_kernel,
        grid=(B, NRB),
        in_specs=[
            pl.BlockSpec((1, 1, 4, ROWS), lambda b, rb: (b, rb, 0, 0)),
            pl.BlockSpec((1, C, H * W), lambda b, rb: (b, 0, 0)),
        ],
        out_specs=pl.BlockSpec((1, 1, C, ROWS), lambda b, rb: (b, rb, 0, 0)),
        out_shape=jax.ShapeDtypeStruct((B, NRB, C, ROWS), jnp.float32),
    )(rois, feat2)
    flat = pooled.reshape(B, NRB, C, RBLK, NBINS)
    flat = jnp.transpose(flat, (0, 1, 3, 2, 4)).reshape(B, RPAD, C * NBINS)
    flat = flat[:, :POST_NMS].reshape(B * POST_NMS, C * NBINS)

    M = B * POST_NMS
    K = flat.shape[1]
    N = W_fc.shape[0]
    BN, BK = 128, 896
    b2 = b_fc.reshape(1, N)
    out = pl.pallas_call(
        _fc_kernel,
        grid=(N // BN, K // BK),
        in_specs=[
            pl.BlockSpec((M, BK), lambda n, k: (0, k)),
            pl.BlockSpec((BN, BK), lambda n, k: (n, k)),
            pl.BlockSpec((1, BN), lambda n, k: (0, n)),
        ],
        out_specs=pl.BlockSpec((M, BN), lambda n, k: (0, n)),
        out_shape=jax.ShapeDtypeStruct((M, N), jnp.float32),
    )(flat, W_fc, b2)
    return out
